# baseline (speedup 1.0000x reference)
import functools

import jax
import jax.numpy as jnp
from jax import lax
from jax.experimental import pallas as pl
from jax.experimental.pallas import tpu as pltpu
from jax.experimental.pallas import tpu_sc as plsc

N = 10000
NP = 10240
E = 320000
DIN = 128
H1, D1 = 8, 8
H2, D2 = 1, 40

NC, NS, L = 2, 16, 16
NW = NC * NS
C = 80
NCHUNK = 125
EPAD = NW * NCHUNK * C
ROWS_PT = NP // NS
ZR = 128

T1W = 80
T2W = 16
T1W2 = 48

def _edge_kernel_body(tw, el_blk, mul_blks, store_ee, bcast_all,
                      t1, t2, src_h, dst_h, out,
                      SI, DI, G, R, S, EB, Z, acc,
                      gs0, gs1, gs2, rs0, rs1, rs2, ss0, ss1, ss2, zs):
    c = lax.axis_index("c")
    s = lax.axis_index("s")
    wid = s * NC + c
    lane = lax.iota(jnp.int32, L)
    zero16 = jnp.zeros((L,), jnp.float32)
    nblk = tw // L

    def zrow(r, carry):
        for j in range(nblk):
            Z[r, pl.ds(L * j, L)] = zero16
        return carry

    lax.fori_loop(0, ZR, zrow, 0)
    row0 = s * ROWS_PT
    for k in range(ROWS_PT // ZR):
        pltpu.async_copy(Z, acc.at[pl.ds(row0 + k * ZR, ZR)], zs)
    pltpu.async_copy(src_h.at[wid], SI, gs0)
    pltpu.async_copy(dst_h.at[wid], DI, rs0)
    for k in range(ROWS_PT // ZR):
        pltpu.make_async_copy(Z, acc.at[pl.ds(row0, ZR)], zs).wait()
    pltpu.make_async_copy(src_h.at[wid], SI, gs0).wait()
    pltpu.make_async_copy(dst_h.at[wid], DI, rs0).wait()
    plsc.subcore_barrier()

    gsems = [gs0, gs1, gs2]
    rsems = [rs0, rs1, rs2]
    ssems = [ss0, ss1, ss2]

    def issue_gather(i, b):
        pltpu.async_copy(t1.at[SI.at[i]], G.at[b], gsems[b])
        pltpu.async_copy(t2.at[DI.at[i]], R.at[b], rsems[b])

    def compute_scatter(i, b):
        pltpu.make_async_copy(t1.at[SI.at[i]], G.at[b], gsems[b]).wait()
        pltpu.make_async_copy(t2.at[DI.at[i]], R.at[b], rsems[b]).wait()
        if isinstance(i, int):
            pltpu.make_async_copy(S.at[b], acc.at[DI.at[i - 3]],
                                  ssems[b]).wait()
        else:
            @pl.when(i >= 3)
            def _wait_prev():
                pltpu.make_async_copy(S.at[b], acc.at[DI.at[i - 3]],
                                      ssems[b]).wait()

        @plsc.parallel_loop(0, C, unroll=8)
        def edge(e):
            elv = G[b, e, pl.ds(L * el_blk, L)]
            erv = R[b, e, pl.ds(0, L)]
            t = elv + erv
            t = jnp.where(t > 0.0, t, 0.2 * t)
            ee = jnp.exp(t)
            EB[e, pl.ds(0, L)] = ee
            erow = lane * 0 + e
            for j in range(mul_blks):
                f = G[b, e, pl.ds(L * j, L)]
                if bcast_all:
                    icol = jnp.where(lane >= 0, 8, 8)
                else:
                    icol = jnp.where(lane >= (L // 2), 2 * j + 1, 2 * j)
                m = plsc.load_gather(EB, [erow, icol])
                S[b, e, pl.ds(L * j, L)] = f * m
            if store_ee:
                S[b, e, pl.ds(L * el_blk, L)] = ee

        pltpu.async_copy(S.at[b], acc.at[DI.at[i]], ssems[b], add=True)

    issue_gather(0, 0)
    issue_gather(1, 1)

    def triple(k3, carry):
        for b in range(3):
            i = 3 * k3 + b
            issue_gather(jnp.minimum(i + 2, NCHUNK - 1), (b + 2) % 3)
            compute_scatter(i, b)
        return carry

    lax.fori_loop(0, NCHUNK // 3, triple, 0)
    for i in range(3 * (NCHUNK // 3), NCHUNK):
        compute_scatter(i, i % 3)

    for b in range(3):
        pltpu.make_async_copy(S.at[b], acc.at[DI.at[0]], ssems[b]).wait()
    plsc.subcore_barrier()
    for k in range(ROWS_PT // ZR):
        r0 = row0 + k * ZR
        pltpu.async_copy(acc.at[pl.ds(r0, ZR)], out.at[c, pl.ds(r0, ZR)], zs)
    for k in range(ROWS_PT // ZR):
        pltpu.make_async_copy(acc.at[pl.ds(row0, ZR)],
                              out.at[c, pl.ds(row0, ZR)], zs).wait()


@functools.cache
def _make_edge_kernel(tw, el_blk, mul_blks, store_ee, bcast_all):
    body = functools.partial(_edge_kernel_body, tw, el_blk, mul_blks,
                             store_ee, bcast_all)
    mesh = plsc.VectorSubcoreMesh(core_axis_name="c", subcore_axis_name="s",
                                  num_cores=NC, num_subcores=NS)
    return pl.kernel(
        body,
        out_type=jax.ShapeDtypeStruct((NC, NP, tw), jnp.float32),
        mesh=mesh,
        scratch_types=[
            pltpu.VMEM((NCHUNK, C), jnp.int32),
            pltpu.VMEM((NCHUNK, C), jnp.int32),
            pltpu.VMEM((3, C, tw), jnp.float32),
            pltpu.VMEM((3, C, T2W), jnp.float32),
            pltpu.VMEM((3, C, tw), jnp.float32),
            pltpu.VMEM((C, L), jnp.float32),
            pltpu.VMEM((ZR, tw), jnp.float32),
            pltpu.VMEM_SHARED((NP, tw), jnp.float32),
            pltpu.SemaphoreType.DMA,
            pltpu.SemaphoreType.DMA,
            pltpu.SemaphoreType.DMA,
            pltpu.SemaphoreType.DMA,
            pltpu.SemaphoreType.DMA,
            pltpu.SemaphoreType.DMA,
            pltpu.SemaphoreType.DMA,
            pltpu.SemaphoreType.DMA,
            pltpu.SemaphoreType.DMA,
            pltpu.SemaphoreType.DMA,
        ],
        compiler_params=pltpu.CompilerParams(use_tc_tiling_on_sc=False,
                                             needs_layout_passes=False),
    )


_TCB = 1000


def _tc1_body(x_ref, w1_ref, alf_ref, arf_ref, t1_ref, t2_ref):
    f1 = H1 * D1
    feat = jnp.dot(x_ref[...], w1_ref[...], preferred_element_type=jnp.float32)
    rows = lax.broadcasted_iota(jnp.int32, (f1, H1), 0)
    cols = lax.broadcasted_iota(jnp.int32, (f1, H1), 1)
    seg = (rows // D1 == cols).astype(jnp.float32)
    el = jnp.dot(feat * alf_ref[...], seg, preferred_element_type=jnp.float32)
    er = jnp.dot(feat * arf_ref[...], seg, preferred_element_type=jnp.float32)
    zpad = jnp.zeros((feat.shape[0], T1W - f1 - H1), jnp.float32)
    t1_ref[...] = jnp.concatenate([feat, el, zpad], axis=-1)
    t2_ref[...] = jnp.concatenate([er, zpad], axis=-1)


def _tc2_body(p_ref, b1_ref, w2_ref, al2_ref, ar2_ref, t1_ref, t2_ref):
    f1 = H1 * D1
    ssum = p_ref[0] + p_ref[1]
    rows = lax.broadcasted_iota(jnp.int32, (H1, f1), 0)
    cols = lax.broadcasted_iota(jnp.int32, (H1, f1), 1)
    seg = (cols // D1 == rows).astype(jnp.float32)
    rden = jnp.dot(1.0 / jnp.maximum(ssum[:, f1:f1 + H1], 1e-9), seg,
                   preferred_element_type=jnp.float32)
    h = ssum[:, :f1] * rden + b1_ref[...]
    h = jnp.where(h > 0.0, h, jnp.exp(h) - 1.0)
    feat2 = jnp.dot(h, w2_ref[...], preferred_element_type=jnp.float32)
    el2 = jnp.dot(feat2, al2_ref[...], preferred_element_type=jnp.float32)
    er2 = jnp.dot(feat2, ar2_ref[...], preferred_element_type=jnp.float32)
    nrow = feat2.shape[0]
    one = jnp.ones((nrow, 1), jnp.float32)
    z6 = jnp.zeros((nrow, T1W2 - D2 - 2), jnp.float32)
    t1_ref[...] = jnp.concatenate([feat2, el2, one, z6], axis=-1)
    z8 = jnp.zeros((nrow, 8), jnp.float32)
    z7 = jnp.zeros((nrow, 7), jnp.float32)
    t2_ref[...] = jnp.concatenate([z8, er2, z7], axis=-1)


def _tc3_body(p_ref, b2_ref, o_ref):
    ssum = p_ref[0] + p_ref[1]
    den = jnp.maximum(ssum[:, D2 + 1:D2 + 2], 1e-9)
    o_ref[...] = ssum[:, :D2] / den + b2_ref[...]


def _full(shape):
    return pl.BlockSpec(shape, lambda i: (0,) * len(shape))


_tc1 = pl.pallas_call(
    _tc1_body,
    grid=(N // _TCB,),
    in_specs=[pl.BlockSpec((_TCB, DIN), lambda i: (i, 0)),
              _full((DIN, 64)), _full((1, 64)), _full((1, 64))],
    out_specs=[pl.BlockSpec((_TCB, T1W), lambda i: (i, 0)),
               pl.BlockSpec((_TCB, T2W), lambda i: (i, 0))],
    out_shape=[jax.ShapeDtypeStruct((N, T1W), jnp.float32),
               jax.ShapeDtypeStruct((N, T2W), jnp.float32)],
)

_tc2 = pl.pallas_call(
    _tc2_body,
    grid=(N // _TCB,),
    in_specs=[pl.BlockSpec((NC, _TCB, T1W), lambda i: (0, i, 0)),
              _full((1, 64)), _full((64, D2)), _full((D2, 1)),
              _full((D2, 1))],
    out_specs=[pl.BlockSpec((_TCB, T1W2), lambda i: (i, 0)),
               pl.BlockSpec((_TCB, T2W), lambda i: (i, 0))],
    out_shape=[jax.ShapeDtypeStruct((N, T1W2), jnp.float32),
               jax.ShapeDtypeStruct((N, T2W), jnp.float32)],
)

_tc3 = pl.pallas_call(
    _tc3_body,
    grid=(N // _TCB,),
    in_specs=[pl.BlockSpec((NC, _TCB, T1W2), lambda i: (0, i, 0)),
              _full((1, D2))],
    out_specs=pl.BlockSpec((_TCB, D2), lambda i: (i, 0)),
    out_shape=jax.ShapeDtypeStruct((N, D2), jnp.float32),
)


def kernel(x, edge_index, W1, attn_l1, attn_r1, bias1,
           W2, attn_l2, attn_r2, bias2):
    f1 = H1 * D1
    src = edge_index[0].reshape(NW, NCHUNK, C)
    dst = edge_index[1].reshape(NW, NCHUNK, C)

    t1, t2 = _tc1(x, W1, attn_l1.reshape(1, f1), attn_r1.reshape(1, f1))
    p1 = _make_edge_kernel(T1W, 4, 4, True, False)(t1, t2, src, dst)
    t1p, t2p = _tc2(p1, bias1.reshape(1, f1), W2, attn_l2.reshape(D2, 1),
                    attn_r2.reshape(D2, 1))
    p2 = _make_edge_kernel(T1W2, 2, 3, False, True)(t1p, t2p, src, dst)
    return _tc3(p2, bias2.reshape(1, D2))


# speedup vs baseline: 1.0109x; 1.0109x over previous
import functools

import jax
import jax.numpy as jnp
from jax import lax
from jax.experimental import pallas as pl
from jax.experimental.pallas import tpu as pltpu
from jax.experimental.pallas import tpu_sc as plsc

N = 10000
NP = 10240
E = 320000
DIN = 128
H1, D1 = 8, 8
H2, D2 = 1, 40

NC, NS, L = 2, 16, 16
NW = NC * NS
C = 80
NCHUNK = 125
EPAD = NW * NCHUNK * C
ROWS_PT = NP // NS
ZR = 128

T1W = 80
T2W = 16
T1W2 = 48

def _edge_kernel_body(tw, el_blk, mul_blks, store_ee, bcast_all,
                      t1, t2, src_h, dst_h, out,
                      SI, DI, G, R, S, EB, ER, Z, acc,
                      gs0, gs1, gs2, rs0, rs1, rs2, ss0, ss1, ss2, zs):
    c = lax.axis_index("c")
    s = lax.axis_index("s")
    wid = s * NC + c
    lane = lax.iota(jnp.int32, L)
    zero16 = jnp.zeros((L,), jnp.float32)
    nblk = tw // L

    def zrow(r, carry):
        for j in range(nblk):
            Z[r, pl.ds(L * j, L)] = zero16
        return carry

    lax.fori_loop(0, ZR, zrow, 0)
    row0 = s * ROWS_PT
    for k in range(ROWS_PT // ZR):
        pltpu.async_copy(Z, acc.at[pl.ds(row0 + k * ZR, ZR)], zs)
    pltpu.async_copy(src_h.at[wid], SI, gs0)
    pltpu.async_copy(dst_h.at[wid], DI, rs0)
    if bcast_all:
        pltpu.async_copy(t2, ER, rs1)
    for k in range(ROWS_PT // ZR):
        pltpu.make_async_copy(Z, acc.at[pl.ds(row0, ZR)], zs).wait()
    pltpu.make_async_copy(src_h.at[wid], SI, gs0).wait()
    pltpu.make_async_copy(dst_h.at[wid], DI, rs0).wait()
    if bcast_all:
        pltpu.make_async_copy(t2, ER, rs1).wait()
    plsc.subcore_barrier()

    gsems = [gs0, gs1, gs2]
    rsems = [rs0, rs1, rs2]
    ssems = [ss0, ss1, ss2]

    def issue_gather(i, b):
        pltpu.async_copy(t1.at[SI.at[i]], G.at[b], gsems[b])
        if not bcast_all:
            pltpu.async_copy(t2.at[DI.at[i]], R.at[b], rsems[b])

    def compute_scatter(i, b):
        pltpu.make_async_copy(t1.at[SI.at[i]], G.at[b], gsems[b]).wait()
        if not bcast_all:
            pltpu.make_async_copy(t2.at[DI.at[i]], R.at[b], rsems[b]).wait()
        if isinstance(i, int):
            pltpu.make_async_copy(S.at[b], acc.at[DI.at[i - 3]],
                                  ssems[b]).wait()
        else:
            @pl.when(i >= 3)
            def _wait_prev():
                pltpu.make_async_copy(S.at[b], acc.at[DI.at[i - 3]],
                                      ssems[b]).wait()

        @plsc.parallel_loop(0, C, unroll=8)
        def edge(e):
            e16 = lane * 0 + e
            if bcast_all:
                d16 = plsc.load_gather(DI, [lane * 0 + i, e16])
                er16 = plsc.load_gather(ER, [d16])
                el16 = plsc.load_gather(
                    G, [lane * 0 + b, e16, lane * 0 + L * el_blk + 8])
                t = el16 + er16
                t = jnp.where(t > 0.0, t, 0.2 * t)
                ee = jnp.exp(t)
                for j in range(mul_blks):
                    S[b, e, pl.ds(L * j, L)] = G[b, e, pl.ds(L * j, L)] * ee
            else:
                elv = G[b, e, pl.ds(L * el_blk, L)]
                erv = R[b, e, pl.ds(0, L)]
                t = elv + erv
                t = jnp.where(t > 0.0, t, 0.2 * t)
                ee = jnp.exp(t)
                EB[e, pl.ds(0, L)] = ee
                for j in range(mul_blks):
                    f = G[b, e, pl.ds(L * j, L)]
                    icol = jnp.where(lane >= (L // 2), 2 * j + 1, 2 * j)
                    m = plsc.load_gather(EB, [e16, icol])
                    S[b, e, pl.ds(L * j, L)] = f * m
                if store_ee:
                    S[b, e, pl.ds(L * el_blk, L)] = ee

        pltpu.async_copy(S.at[b], acc.at[DI.at[i]], ssems[b], add=True)

    issue_gather(0, 0)
    issue_gather(1, 1)

    def triple(k3, carry):
        for b in range(3):
            i = 3 * k3 + b
            issue_gather(jnp.minimum(i + 2, NCHUNK - 1), (b + 2) % 3)
            compute_scatter(i, b)
        return carry

    lax.fori_loop(0, NCHUNK // 3, triple, 0)
    for i in range(3 * (NCHUNK // 3), NCHUNK):
        compute_scatter(i, i % 3)

    for b in range(3):
        pltpu.make_async_copy(S.at[b], acc.at[DI.at[0]], ssems[b]).wait()
    plsc.subcore_barrier()
    for k in range(ROWS_PT // ZR):
        r0 = row0 + k * ZR
        pltpu.async_copy(acc.at[pl.ds(r0, ZR)], out.at[c, pl.ds(r0, ZR)], zs)
    for k in range(ROWS_PT // ZR):
        pltpu.make_async_copy(acc.at[pl.ds(row0, ZR)],
                              out.at[c, pl.ds(row0, ZR)], zs).wait()


@functools.cache
def _make_edge_kernel(tw, el_blk, mul_blks, store_ee, bcast_all):
    body = functools.partial(_edge_kernel_body, tw, el_blk, mul_blks,
                             store_ee, bcast_all)
    mesh = plsc.VectorSubcoreMesh(core_axis_name="c", subcore_axis_name="s",
                                  num_cores=NC, num_subcores=NS)
    return pl.kernel(
        body,
        out_type=jax.ShapeDtypeStruct((NC, NP, tw), jnp.float32),
        mesh=mesh,
        scratch_types=[
            pltpu.VMEM((NCHUNK, C), jnp.int32),
            pltpu.VMEM((NCHUNK, C), jnp.int32),
            pltpu.VMEM((3, C, tw), jnp.float32),
            pltpu.VMEM((3, C, T2W), jnp.float32),
            pltpu.VMEM((3, C, tw), jnp.float32),
            pltpu.VMEM((C, L), jnp.float32),
            pltpu.VMEM((N,) if bcast_all else (L,), jnp.float32),
            pltpu.VMEM((ZR, tw), jnp.float32),
            pltpu.VMEM_SHARED((NP, tw), jnp.float32),
            pltpu.SemaphoreType.DMA,
            pltpu.SemaphoreType.DMA,
            pltpu.SemaphoreType.DMA,
            pltpu.SemaphoreType.DMA,
            pltpu.SemaphoreType.DMA,
            pltpu.SemaphoreType.DMA,
            pltpu.SemaphoreType.DMA,
            pltpu.SemaphoreType.DMA,
            pltpu.SemaphoreType.DMA,
            pltpu.SemaphoreType.DMA,
        ],
        compiler_params=pltpu.CompilerParams(use_tc_tiling_on_sc=False,
                                             needs_layout_passes=False),
    )


_TCB = 1000


def _tc1_body(x_ref, w1_ref, alf_ref, arf_ref, t1_ref, t2_ref):
    f1 = H1 * D1
    feat = jnp.dot(x_ref[...], w1_ref[...], preferred_element_type=jnp.float32)
    rows = lax.broadcasted_iota(jnp.int32, (f1, H1), 0)
    cols = lax.broadcasted_iota(jnp.int32, (f1, H1), 1)
    seg = (rows // D1 == cols).astype(jnp.float32)
    el = jnp.dot(feat * alf_ref[...], seg, preferred_element_type=jnp.float32)
    er = jnp.dot(feat * arf_ref[...], seg, preferred_element_type=jnp.float32)
    zpad = jnp.zeros((feat.shape[0], T1W - f1 - H1), jnp.float32)
    t1_ref[...] = jnp.concatenate([feat, el, zpad], axis=-1)
    t2_ref[...] = jnp.concatenate([er, zpad], axis=-1)


def _tc2_body(p_ref, b1_ref, w2_ref, al2_ref, ar2_ref, t1_ref, t2_ref):
    f1 = H1 * D1
    ssum = p_ref[0] + p_ref[1]
    rows = lax.broadcasted_iota(jnp.int32, (H1, f1), 0)
    cols = lax.broadcasted_iota(jnp.int32, (H1, f1), 1)
    seg = (cols // D1 == rows).astype(jnp.float32)
    rden = jnp.dot(1.0 / jnp.maximum(ssum[:, f1:f1 + H1], 1e-9), seg,
                   preferred_element_type=jnp.float32)
    h = ssum[:, :f1] * rden + b1_ref[...]
    h = jnp.where(h > 0.0, h, jnp.exp(h) - 1.0)
    feat2 = jnp.dot(h, w2_ref[...], preferred_element_type=jnp.float32)
    el2 = jnp.dot(feat2, al2_ref[...], preferred_element_type=jnp.float32)
    er2 = jnp.dot(feat2, ar2_ref[...], preferred_element_type=jnp.float32)
    nrow = feat2.shape[0]
    one = jnp.ones((nrow, 1), jnp.float32)
    z6 = jnp.zeros((nrow, T1W2 - D2 - 2), jnp.float32)
    t1_ref[...] = jnp.concatenate([feat2, el2, one, z6], axis=-1)
    t2_ref[...] = er2


def _tc3_body(p_ref, b2_ref, o_ref):
    ssum = p_ref[0] + p_ref[1]
    den = jnp.maximum(ssum[:, D2 + 1:D2 + 2], 1e-9)
    o_ref[...] = ssum[:, :D2] / den + b2_ref[...]


def _full(shape):
    return pl.BlockSpec(shape, lambda i: (0,) * len(shape))


_tc1 = pl.pallas_call(
    _tc1_body,
    grid=(N // _TCB,),
    in_specs=[pl.BlockSpec((_TCB, DIN), lambda i: (i, 0)),
              _full((DIN, 64)), _full((1, 64)), _full((1, 64))],
    out_specs=[pl.BlockSpec((_TCB, T1W), lambda i: (i, 0)),
               pl.BlockSpec((_TCB, T2W), lambda i: (i, 0))],
    out_shape=[jax.ShapeDtypeStruct((N, T1W), jnp.float32),
               jax.ShapeDtypeStruct((N, T2W), jnp.float32)],
)

_tc2 = pl.pallas_call(
    _tc2_body,
    grid=(N // _TCB,),
    in_specs=[pl.BlockSpec((NC, _TCB, T1W), lambda i: (0, i, 0)),
              _full((1, 64)), _full((64, D2)), _full((D2, 1)),
              _full((D2, 1))],
    out_specs=[pl.BlockSpec((_TCB, T1W2), lambda i: (i, 0)),
               pl.BlockSpec((_TCB, 1), lambda i: (i, 0))],
    out_shape=[jax.ShapeDtypeStruct((N, T1W2), jnp.float32),
               jax.ShapeDtypeStruct((N, 1), jnp.float32)],
)

_tc3 = pl.pallas_call(
    _tc3_body,
    grid=(N // _TCB,),
    in_specs=[pl.BlockSpec((NC, _TCB, T1W2), lambda i: (0, i, 0)),
              _full((1, D2))],
    out_specs=pl.BlockSpec((_TCB, D2), lambda i: (i, 0)),
    out_shape=jax.ShapeDtypeStruct((N, D2), jnp.float32),
)


def kernel(x, edge_index, W1, attn_l1, attn_r1, bias1,
           W2, attn_l2, attn_r2, bias2):
    f1 = H1 * D1
    src = edge_index[0].reshape(NW, NCHUNK, C)
    dst = edge_index[1].reshape(NW, NCHUNK, C)

    t1, t2 = _tc1(x, W1, attn_l1.reshape(1, f1), attn_r1.reshape(1, f1))
    p1 = _make_edge_kernel(T1W, 4, 4, True, False)(t1, t2, src, dst)
    t1p, t2p = _tc2(p1, bias1.reshape(1, f1), W2, attn_l2.reshape(D2, 1),
                    attn_r2.reshape(D2, 1))
    p2 = _make_edge_kernel(T1W2, 2, 3, False, True)(
        t1p, t2p.reshape(N), src, dst)
    return _tc3(p2, bias2.reshape(1, D2))
